# baseline (speedup 1.0000x reference)
import jax
import jax.numpy as jnp
from jax.experimental import pallas as pl
from jax.experimental.pallas import tpu as pltpu

_NUM_POSITION = 64
_DECLAY = 0.4
_NUM_HIERARCHY = 3
_LAST_P = int(_NUM_POSITION * _DECLAY ** (_NUM_HIERARCHY - 1))


_NB = 2


def _reduce_kernel(x_ref, o_ref):
    m_scale = 1.0 / (2.0 * _LAST_P)
    o_ref[:, 0, :] = jnp.sum(x_ref[...], axis=1) * m_scale


def kernel(x, protos0, protos1, protos2):
    N, M, C, T, V = x.shape
    assert M == 2 and N % _NB == 0
    xt = jnp.transpose(x, (0, 1, 4, 3, 2)).reshape(N, M * V * T, C)
    part = pl.pallas_call(
        _reduce_kernel,
        grid=(N // _NB,),
        in_specs=[pl.BlockSpec((_NB, M * V * T, C), lambda i: (i, 0, 0))],
        out_specs=pl.BlockSpec((_NB, 1, C), lambda i: (i, 0, 0)),
        out_shape=jax.ShapeDtypeStruct((N, 1, C), jnp.float32),
        compiler_params=pltpu.CompilerParams(dimension_semantics=("parallel",)),
    )(xt)
    return part.reshape(N, C)


# speedup vs baseline: 1.0111x; 1.0111x over previous
import jax
import jax.numpy as jnp
from jax.experimental import pallas as pl

_NUM_POSITION = 64
_DECLAY = 0.4
_NUM_HIERARCHY = 3
_LAST_P = int(_NUM_POSITION * _DECLAY ** (_NUM_HIERARCHY - 1))

_NB = 8


def _reduce_kernel(x_ref, o_ref):
    m_scale = 1.0 / (2.0 * _LAST_P)
    i = pl.program_id(0)
    o_ref[pl.ds(i * _NB, _NB), :] = jnp.sum(x_ref[...], axis=1) * m_scale


def kernel(x, protos0, protos1, protos2):
    N, M, C, T, V = x.shape
    assert M == 2 and N % _NB == 0
    xt = jnp.transpose(x, (0, 1, 4, 3, 2)).reshape(N, M * V * T, C)
    return pl.pallas_call(
        _reduce_kernel,
        grid=(N // _NB,),
        in_specs=[pl.BlockSpec((_NB, M * V * T, C), lambda i: (i, 0, 0))],
        out_specs=pl.BlockSpec((N, C), lambda i: (0, 0)),
        out_shape=jax.ShapeDtypeStruct((N, C), jnp.float32),
    )(xt)
